# baseline (speedup 1.0000x reference)
import jax
import jax.numpy as jnp
from jax import lax
from jax.experimental import pallas as pl
from jax.experimental.pallas import tpu as pltpu
from jax.experimental.pallas import tpu_sc as plsc

_NB = 999999
_K = 0x45D9F3B
_ML = 35747
_MH = 3183
_N = 16384 * 200
_NW = 32
_PW = _N // _NW


def _lshr(v, k):
    return lax.shift_right_logical(v, jnp.int32(k))


def _hash_mod(v):
    v = v ^ _lshr(v, 16)
    v = v * jnp.int32(_K)
    v = v ^ _lshr(v, 16)
    v = v * jnp.int32(_K)
    v = v ^ _lshr(v, 16)
    xl = v & jnp.int32(0xFFFF)
    xh = _lshr(v, 16)
    lo = xl * jnp.int32(_ML)
    t1 = xh * jnp.int32(_ML) + _lshr(lo, 16)
    u = xl * jnp.int32(_MH) + (t1 & jnp.int32(0xFFFF))
    hi = xh * jnp.int32(_MH) + _lshr(t1, 16) + _lshr(u, 16)
    q = _lshr(_lshr(v - hi, 1) + hi, 19)
    return v - q * jnp.int32(_NB)


_NCH = 2
_CW = _PW // _NCH
_U = 8


def _body(x_hbm, o_hbm, ibuf, obuf):
    wid = lax.axis_index("s") * 2 + lax.axis_index("c")
    base = wid * _PW

    def one_chunk(c):
        off = base + c * _CW
        pltpu.sync_copy(x_hbm.at[pl.ds(off, _CW)], ibuf)

        @plsc.parallel_loop(0, _CW, 16, unroll=_U)
        def _(i):
            v = ibuf[pl.ds(i, 16)]
            h = _hash_mod(v)
            obuf[pl.ds(i, 16)] = jnp.where(
                v != 0, h + jnp.int32(1), jnp.int32(0)
            )

        pltpu.sync_copy(obuf, o_hbm.at[pl.ds(off, _CW)])

    for c in range(_NCH):
        one_chunk(c)


def kernel(x):
    xf = x.reshape(_N)
    run = pl.kernel(
        _body,
        out_type=jax.ShapeDtypeStruct((_N,), jnp.int32),
        mesh=plsc.VectorSubcoreMesh(core_axis_name="c", subcore_axis_name="s"),
        scratch_types=[
            pltpu.VMEM((_CW,), jnp.int32),
            pltpu.VMEM((_CW,), jnp.int32),
        ],
    )
    return run(xf).reshape(x.shape)


# speedup vs baseline: 1.0967x; 1.0967x over previous
import jax
import jax.numpy as jnp
from jax import lax
from jax.experimental import pallas as pl
from jax.experimental.pallas import tpu as pltpu
from jax.experimental.pallas import tpu_sc as plsc

_NB = 999999
_K = 0x45D9F3B
_ML = 35747
_MH = 3183
_N = 16384 * 200
_NW = 32
_PW = _N // _NW


def _lshr(v, k):
    return lax.shift_right_logical(v, jnp.int32(k))


def _hash_mod(v):
    v = v ^ _lshr(v, 16)
    v = v * jnp.int32(_K)
    v = v ^ _lshr(v, 16)
    v = v * jnp.int32(_K)
    v = v ^ _lshr(v, 16)
    q0 = _lshr(_lshr(v, 16) * jnp.int32(4295), 16)
    r = v - q0 * jnp.int32(_NB)
    r = jnp.where(r >= jnp.int32(_NB), r - jnp.int32(_NB), r)
    r = jnp.where(r < 0, r + jnp.int32(_NB), r)
    return r


_NCH = 2
_CW = _PW // _NCH
_U = 8


def _body(x_hbm, o_hbm, ibuf, obuf):
    wid = lax.axis_index("s") * 2 + lax.axis_index("c")
    base = wid * _PW

    def one_chunk(c):
        off = base + c * _CW
        pltpu.sync_copy(x_hbm.at[pl.ds(off, _CW)], ibuf)

        @plsc.parallel_loop(0, _CW, 16, unroll=_U)
        def _(i):
            v = ibuf[pl.ds(i, 16)]
            h = _hash_mod(v)
            obuf[pl.ds(i, 16)] = jnp.where(
                v != 0, h + jnp.int32(1), jnp.int32(0)
            )

        pltpu.sync_copy(obuf, o_hbm.at[pl.ds(off, _CW)])

    for c in range(_NCH):
        one_chunk(c)


def kernel(x):
    xf = x.reshape(_N)
    run = pl.kernel(
        _body,
        out_type=jax.ShapeDtypeStruct((_N,), jnp.int32),
        mesh=plsc.VectorSubcoreMesh(core_axis_name="c", subcore_axis_name="s"),
        scratch_types=[
            pltpu.VMEM((_CW,), jnp.int32),
            pltpu.VMEM((_CW,), jnp.int32),
        ],
    )
    return run(xf).reshape(x.shape)


# speedup vs baseline: 1.4321x; 1.3058x over previous
import jax
import jax.numpy as jnp
from jax import lax
from jax.experimental import pallas as pl
from jax.experimental.pallas import tpu as pltpu
from jax.experimental.pallas import tpu_sc as plsc

_NB = 999999
_K = 0x45D9F3B
_R = 16384
_C = 200
_NW = 32
_RW = _R // _NW
_NCH = 4
_CR = _RW // _NCH
_NV = _C // 16
_TAIL = _C - 16


def _lshr(v, k):
    return lax.shift_right_logical(v, jnp.int32(k))


def _hash_mod(v):
    v = v ^ _lshr(v, 16)
    v = v * jnp.int32(_K)
    v = v ^ _lshr(v, 16)
    v = v * jnp.int32(_K)
    v = v ^ _lshr(v, 16)
    q0 = _lshr(_lshr(v, 16) * jnp.int32(4295), 16)
    r = v - q0 * jnp.int32(_NB)
    r = jnp.where(r >= jnp.int32(_NB), r - jnp.int32(_NB), r)
    r = jnp.where(r < 0, r + jnp.int32(_NB), r)
    return r


def _bucketize(v):
    h = _hash_mod(v)
    return jnp.where(v != 0, h + jnp.int32(1), jnp.int32(0))


def _body(x_hbm, o_hbm, ibuf, obuf):
    wid = lax.axis_index("s") * 2 + lax.axis_index("c")
    base = wid * _RW

    def one_chunk(c):
        r0 = base + c * _CR
        pltpu.sync_copy(x_hbm.at[pl.ds(r0, _CR)], ibuf)

        @plsc.parallel_loop(0, _CR, 1, unroll=2)
        def _(r):
            for j in range(_NV):
                v = ibuf[r, pl.ds(j * 16, 16)]
                obuf[r, pl.ds(j * 16, 16)] = _bucketize(v)
            v = ibuf[r, pl.ds(_TAIL, 16)]
            obuf[r, pl.ds(_TAIL, 16)] = _bucketize(v)

        pltpu.sync_copy(obuf, o_hbm.at[pl.ds(r0, _CR)])

    for c in range(_NCH):
        one_chunk(c)


def kernel(x):
    run = pl.kernel(
        _body,
        out_type=jax.ShapeDtypeStruct((_R, _C), jnp.int32),
        mesh=plsc.VectorSubcoreMesh(core_axis_name="c", subcore_axis_name="s"),
        scratch_types=[
            pltpu.VMEM((_CR, _C), jnp.int32),
            pltpu.VMEM((_CR, _C), jnp.int32),
        ],
    )
    return run(x)
